# Initial kernel scaffold; baseline (speedup 1.0000x reference)
import functools

import jax
import jax.numpy as jnp
from jax import lax
from jax.experimental import pallas as pl
from jax.experimental.pallas import tpu as pltpu
from jax.experimental.pallas import tpu_sc as plsc

N, E, D, H, C, G = 10000, 320000, 128, 16, 8, 64
NC, NS = 2, 16
NW = NC * NS
CH = 128
NPAD = 10112
EPW = NPAD
NCHUNK = EPW // CH
ROWS_PER_TILE = NPAD // NS
EPAD = EPW * NW

_MESH = plsc.VectorSubcoreMesh(
    core_axis_name="c", subcore_axis_name="s", num_cores=NC, num_subcores=NS
)


@functools.partial(
    pl.kernel,
    out_type=jax.ShapeDtypeStruct((NC, NPAD), jnp.float32),
    mesh=_MESH,
    scratch_types=[
        pltpu.VMEM_SHARED((NPAD,), jnp.float32),
        pltpu.VMEM((NCHUNK, CH), jnp.int32),
        pltpu.VMEM((CH,), jnp.float32),
    ],
)
def _deg_kernel(dst2_hbm, zeros_hbm, deg_out, deg_sh, didx, ones_v):
    c = lax.axis_index("c")
    s = lax.axis_index("s")
    w = c * NS + s
    r0 = s * ROWS_PER_TILE
    pltpu.sync_copy(
        zeros_hbm.at[pl.ds(r0, ROWS_PER_TILE)], deg_sh.at[pl.ds(r0, ROWS_PER_TILE)]
    )
    pltpu.sync_copy(dst2_hbm.at[pl.ds(w * NCHUNK, NCHUNK)], didx)
    for i in range(CH // 16):
        ones_v[pl.ds(i * 16, 16)] = jnp.full((16,), 1.0, jnp.float32)
    plsc.subcore_barrier()

    def body(j, carry):
        pltpu.sync_copy(ones_v, deg_sh.at[didx.at[j]], add=True)
        return carry

    lax.fori_loop(0, NCHUNK, body, 0)
    plsc.subcore_barrier()
    pltpu.sync_copy(
        deg_sh.at[pl.ds(r0, ROWS_PER_TILE)], deg_out.at[c, pl.ds(r0, ROWS_PER_TILE)]
    )


@functools.partial(
    pl.kernel,
    out_type=jax.ShapeDtypeStruct((NC, NPAD, H), jnp.float32),
    mesh=_MESH,
    scratch_types=[
        pltpu.VMEM_SHARED((NPAD, H), jnp.float32),
        pltpu.VMEM((NCHUNK, CH), jnp.int32),
        pltpu.VMEM((NCHUNK, CH), jnp.int32),
        pltpu.VMEM((CH, H), jnp.float32),
        pltpu.SemaphoreType.DMA,
    ],
)
def _agg_kernel(y_hbm, src2_hbm, dst2_hbm, zeros_hbm, agg_out, agg_sh, sidx, didx, rows, sem):
    c = lax.axis_index("c")
    s = lax.axis_index("s")
    w = c * NS + s
    r0 = s * ROWS_PER_TILE
    pltpu.sync_copy(
        zeros_hbm.at[pl.ds(r0, ROWS_PER_TILE)], agg_sh.at[pl.ds(r0, ROWS_PER_TILE)]
    )
    pltpu.sync_copy(src2_hbm.at[pl.ds(w * NCHUNK, NCHUNK)], sidx)
    pltpu.sync_copy(dst2_hbm.at[pl.ds(w * NCHUNK, NCHUNK)], didx)
    plsc.subcore_barrier()

    def body(j, carry):
        pltpu.async_copy(y_hbm.at[sidx.at[j]], rows, sem).wait()
        pltpu.sync_copy(rows, agg_sh.at[didx.at[j]], add=True)
        return carry

    lax.fori_loop(0, NCHUNK, body, 0)
    plsc.subcore_barrier()
    pltpu.sync_copy(
        agg_sh.at[pl.ds(r0, ROWS_PER_TILE)], agg_out.at[c, pl.ds(r0, ROWS_PER_TILE)]
    )


_PREC = lax.Precision.HIGHEST


def _tc1_body(x_ref, w1_ref, d0_ref, d1_ref, y1_ref, dinv_ref):
    deg = d0_ref[...] + d1_ref[...] + 1.0
    dinv = lax.rsqrt(deg)
    xw = jnp.dot(x_ref[...], w1_ref[...], precision=_PREC,
                 preferred_element_type=jnp.float32)
    dn = dinv[:N]
    y1_ref[...] = xw * dn
    dinv_ref[...] = dn


def _tc2_body(aggp_ref, y1_ref, dinv_ref, b1_ref, w2_ref, y2_ref):
    aggp = aggp_ref[...]
    agg = aggp[0] + aggp[1]
    t = dinv_ref[...] * (agg[:N] + y1_ref[...]) + b1_ref[...]
    h = jnp.maximum(t, 0.0)
    y2_ref[...] = dinv_ref[...] * jnp.dot(
        h, w2_ref[...], precision=_PREC, preferred_element_type=jnp.float32
    )


def _tc3_body(aggp_ref, y2_ref, dinv_ref, b2_ref, batch_ref, out_ref):
    aggp = aggp_ref[...]
    agg = aggp[0] + aggp[1]
    t = dinv_ref[...] * (agg[:N] + y2_ref[...])
    h2 = t[:, :C] + b2_ref[...]
    b = batch_ref[...]
    gids = lax.broadcasted_iota(jnp.int32, (G, 1), 0)
    onehot = (b == gids).astype(jnp.float32)
    sums = jnp.dot(onehot, h2, precision=_PREC,
                   preferred_element_type=jnp.float32)
    counts = jnp.sum(onehot, axis=1, keepdims=True)
    pooled = sums / jnp.maximum(counts, 1.0)
    m = jnp.max(pooled, axis=1, keepdims=True)
    lse = m + jnp.log(jnp.sum(jnp.exp(pooled - m), axis=1, keepdims=True))
    out_ref[...] = pooled - lse


_tc1 = pl.pallas_call(
    _tc1_body,
    out_shape=[
        jax.ShapeDtypeStruct((N, H), jnp.float32),
        jax.ShapeDtypeStruct((N, 1), jnp.float32),
    ],
)

_tc2 = pl.pallas_call(
    _tc2_body,
    out_shape=jax.ShapeDtypeStruct((N, H), jnp.float32),
)

_tc3 = pl.pallas_call(
    _tc3_body,
    out_shape=jax.ShapeDtypeStruct((G, C), jnp.float32),
)


def kernel(x, edge_index, batch, W1, b1, W2, b2):
    src = edge_index[0]
    dst = edge_index[1]
    pad_e = EPAD - E
    pad_src = (jnp.arange(pad_e, dtype=jnp.int32) * 37) % N
    pad_dst = N + (jnp.arange(pad_e, dtype=jnp.int32) % (NPAD - N))
    src2 = jnp.concatenate([src, pad_src]).reshape(NW * NCHUNK, CH)
    dst2 = jnp.concatenate([dst, pad_dst]).reshape(NW * NCHUNK, CH)
    zeros1 = jnp.zeros((NPAD,), jnp.float32)
    zeros2 = jnp.zeros((NPAD, H), jnp.float32)
    w2pad = jnp.concatenate([W2, jnp.zeros((H, H - C), jnp.float32)], axis=1)

    degp = _deg_kernel(dst2, zeros1)
    d0 = degp[0].reshape(NPAD, 1)
    d1 = degp[1].reshape(NPAD, 1)
    y1, dinv = _tc1(x, W1, d0, d1)
    agg1 = _agg_kernel(y1, src2, dst2, zeros2)
    y2 = _tc2(agg1, y1, dinv, b1.reshape(1, H), w2pad)
    agg2 = _agg_kernel(y2, src2, dst2, zeros2)
    out = _tc3(agg2, y2, dinv, b2.reshape(1, C), batch.reshape(1, N))
    return out


# speedup vs baseline: 34.6709x; 34.6709x over previous
import functools

import jax
import jax.numpy as jnp
from jax import lax
from jax.experimental import pallas as pl
from jax.experimental.pallas import tpu as pltpu
from jax.experimental.pallas import tpu_sc as plsc

N, E, D, H, C, G = 10000, 320000, 128, 16, 8, 64
NC, NS = 2, 16
NW = NC * NS
CH = 128
NPAD = 10112
NCHUNK = 80
EPW = NCHUNK * CH
ROWS_PER_TILE = NPAD // NS
EPAD = EPW * NW

def _deg_body(dst2_hbm, zeros_hbm, deg_out, deg_sh, didx, ones_v, bounce):
    c = lax.axis_index("c")
    s = lax.axis_index("s")
    w = c * NS + s
    r0 = s * ROWS_PER_TILE
    pltpu.sync_copy(zeros_hbm, bounce)
    pltpu.sync_copy(bounce, deg_sh.at[pl.ds(r0, ROWS_PER_TILE)])
    pltpu.sync_copy(dst2_hbm.at[pl.ds(w * NCHUNK, NCHUNK)], didx)
    for i in range(CH // 16):
        ones_v[pl.ds(i * 16, 16)] = jnp.full((16,), 1.0, jnp.float32)
    plsc.subcore_barrier()

    def body(j, carry):
        pltpu.sync_copy(ones_v, deg_sh.at[didx.at[j]], add=True)
        return carry

    lax.fori_loop(0, NCHUNK, body, 0)
    plsc.subcore_barrier()
    pltpu.sync_copy(deg_sh.at[pl.ds(r0, ROWS_PER_TILE)], bounce)
    pltpu.sync_copy(bounce, deg_out.at[pl.ds(c * NPAD + r0, ROWS_PER_TILE)])


def _agg_body(y_hbm, src2_hbm, dst2_hbm, zeros_hbm, agg_out, agg_sh, sidx, didx, rows, bounce, sem):
    c = lax.axis_index("c")
    s = lax.axis_index("s")
    w = c * NS + s
    r0 = s * ROWS_PER_TILE
    pltpu.sync_copy(zeros_hbm, bounce)
    pltpu.sync_copy(bounce, agg_sh.at[pl.ds(r0, ROWS_PER_TILE)])
    pltpu.sync_copy(src2_hbm.at[pl.ds(w * NCHUNK, NCHUNK)], sidx)
    pltpu.sync_copy(dst2_hbm.at[pl.ds(w * NCHUNK, NCHUNK)], didx)
    plsc.subcore_barrier()

    def body(j, carry):
        pltpu.async_copy(y_hbm.at[sidx.at[j]], rows, sem).wait()
        pltpu.sync_copy(rows, agg_sh.at[didx.at[j]], add=True)
        return carry

    lax.fori_loop(0, NCHUNK, body, 0)
    plsc.subcore_barrier()
    pltpu.sync_copy(agg_sh.at[pl.ds(r0, ROWS_PER_TILE)], bounce)
    pltpu.sync_copy(bounce, agg_out.at[c, pl.ds(r0, ROWS_PER_TILE)])


@functools.cache
def _sc_kernels():
    mesh = plsc.VectorSubcoreMesh(
        core_axis_name="c", subcore_axis_name="s", num_cores=NC, num_subcores=NS
    )
    params = pltpu.CompilerParams(use_tc_tiling_on_sc=False)
    deg_kernel = pl.kernel(
        _deg_body,
        out_type=jax.ShapeDtypeStruct((NC * NPAD,), jnp.float32),
        mesh=mesh,
        scratch_types=[
            pltpu.VMEM_SHARED((NPAD,), jnp.float32),
            pltpu.VMEM((NCHUNK, CH), jnp.int32),
            pltpu.VMEM((CH,), jnp.float32),
            pltpu.VMEM((ROWS_PER_TILE,), jnp.float32),
        ],
        compiler_params=params,
    )
    agg_kernel = pl.kernel(
        _agg_body,
        out_type=jax.ShapeDtypeStruct((NC, NPAD, H), jnp.float32),
        mesh=mesh,
        scratch_types=[
            pltpu.VMEM_SHARED((NPAD, H), jnp.float32),
            pltpu.VMEM((NCHUNK, CH), jnp.int32),
            pltpu.VMEM((NCHUNK, CH), jnp.int32),
            pltpu.VMEM((CH, H), jnp.float32),
            pltpu.VMEM((ROWS_PER_TILE, H), jnp.float32),
            pltpu.SemaphoreType.DMA,
        ],
        compiler_params=params,
    )
    return deg_kernel, agg_kernel


_PREC = lax.Precision.HIGHEST


def _tc1_body(x_ref, w1_ref, d0_ref, d1_ref, y1_ref, dinv_ref):
    deg = d0_ref[...] + d1_ref[...] + 1.0
    dinv = lax.rsqrt(deg)
    xw = jnp.dot(x_ref[...], w1_ref[...], precision=_PREC,
                 preferred_element_type=jnp.float32)
    dn = dinv[:N]
    y1_ref[...] = xw * dn
    dinv_ref[...] = dn


def _tc2_body(aggp_ref, y1_ref, dinv_ref, b1_ref, w2_ref, y2_ref):
    aggp = aggp_ref[...]
    agg = aggp[0] + aggp[1]
    t = dinv_ref[...] * (agg[:N] + y1_ref[...]) + b1_ref[...]
    h = jnp.maximum(t, 0.0)
    y2_ref[...] = dinv_ref[...] * jnp.dot(
        h, w2_ref[...], precision=_PREC, preferred_element_type=jnp.float32
    )


def _tc3_body(aggp_ref, y2_ref, dinv_ref, b2_ref, batch_ref, out_ref):
    aggp = aggp_ref[...]
    agg = aggp[0] + aggp[1]
    t = dinv_ref[...] * (agg[:N] + y2_ref[...])
    h2 = t[:, :C] + b2_ref[...]
    b = batch_ref[...]
    gids = lax.broadcasted_iota(jnp.int32, (G, 1), 0)
    onehot = (b == gids).astype(jnp.float32)
    sums = jnp.dot(onehot, h2, precision=_PREC,
                   preferred_element_type=jnp.float32)
    counts = jnp.sum(onehot, axis=1, keepdims=True)
    pooled = sums / jnp.maximum(counts, 1.0)
    m = jnp.max(pooled, axis=1, keepdims=True)
    lse = m + jnp.log(jnp.sum(jnp.exp(pooled - m), axis=1, keepdims=True))
    out_ref[...] = pooled - lse


_tc1 = pl.pallas_call(
    _tc1_body,
    out_shape=[
        jax.ShapeDtypeStruct((N, H), jnp.float32),
        jax.ShapeDtypeStruct((N, 1), jnp.float32),
    ],
)

_tc2 = pl.pallas_call(
    _tc2_body,
    out_shape=jax.ShapeDtypeStruct((N, H), jnp.float32),
)

_tc3 = pl.pallas_call(
    _tc3_body,
    out_shape=jax.ShapeDtypeStruct((G, C), jnp.float32),
)


def kernel(x, edge_index, batch, W1, b1, W2, b2):
    src = edge_index[0]
    dst = edge_index[1]
    pad_e = EPAD - E
    pad_src = (jnp.arange(pad_e, dtype=jnp.int32) * 37) % N
    pad_dst = N + (jnp.arange(pad_e, dtype=jnp.int32) % (NPAD - N))
    src2 = jnp.concatenate([src, pad_src]).reshape(NW * NCHUNK, CH)
    dst2 = jnp.concatenate([dst, pad_dst]).reshape(NW * NCHUNK, CH)
    zeros1 = jnp.zeros((ROWS_PER_TILE,), jnp.float32)
    zeros2 = jnp.zeros((ROWS_PER_TILE, H), jnp.float32)
    w2pad = jnp.concatenate([W2, jnp.zeros((H, H - C), jnp.float32)], axis=1)

    _deg_kernel, _agg_kernel = _sc_kernels()
    degp = _deg_kernel(dst2, zeros1)
    d0 = degp[:NPAD].reshape(NPAD, 1)
    d1 = degp[NPAD:].reshape(NPAD, 1)
    y1, dinv = _tc1(x, W1, d0, d1)
    agg1 = _agg_kernel(y1, src2, dst2, zeros2)
    y2 = _tc2(agg1, y1, dinv, b1.reshape(1, H), w2pad)
    agg2 = _agg_kernel(y2, src2, dst2, zeros2)
    out = _tc3(agg2, y2, dinv, b2.reshape(1, C), batch.reshape(1, N))
    return out


# speedup vs baseline: 38.0993x; 1.0989x over previous
import functools

import jax
import jax.numpy as jnp
from jax import lax
from jax.experimental import pallas as pl
from jax.experimental.pallas import tpu as pltpu
from jax.experimental.pallas import tpu_sc as plsc

N, E, D, H, C, G = 10000, 320000, 128, 16, 8, 64
NC, NS = 2, 16
NW = NC * NS
CH = 128
NPAD = 10112
NCHUNK = 80
EPW = NCHUNK * CH
ROWS_PER_TILE = NPAD // NS
EPAD = EPW * NW

def _deg_body(dst2_hbm, zeros_hbm, deg_out, deg_sh, didx, ones_v, bounce, dsem):
    c = lax.axis_index("c")
    s = lax.axis_index("s")
    w = c * NS + s
    r0 = s * ROWS_PER_TILE
    pltpu.sync_copy(zeros_hbm, bounce)
    pltpu.sync_copy(bounce, deg_sh.at[pl.ds(r0, ROWS_PER_TILE)])
    pltpu.sync_copy(dst2_hbm.at[pl.ds(w * NCHUNK, NCHUNK)], didx)
    for i in range(CH // 16):
        ones_v[pl.ds(i * 16, 16)] = jnp.full((16,), 1.0, jnp.float32)
    plsc.subcore_barrier()

    def body(g, carry):
        j0 = g * 4
        for k in range(4):
            pltpu.async_copy(ones_v, deg_sh.at[didx.at[j0 + k]], dsem, add=True)
        for k in range(4):
            pltpu.make_async_copy(ones_v, deg_sh.at[didx.at[j0 + k]], dsem).wait()
        return carry

    lax.fori_loop(0, NCHUNK // 4, body, 0)
    plsc.subcore_barrier()
    pltpu.sync_copy(deg_sh.at[pl.ds(r0, ROWS_PER_TILE)], bounce)
    pltpu.sync_copy(bounce, deg_out.at[pl.ds(c * NPAD + r0, ROWS_PER_TILE)])


def _agg_body(y_hbm, src2_hbm, dst2_hbm, zeros_hbm, agg_out, agg_sh, sidx, didx,
              rows0, rows1, bounce, sem0, sem1):
    c = lax.axis_index("c")
    s = lax.axis_index("s")
    w = c * NS + s
    r0 = s * ROWS_PER_TILE
    pltpu.sync_copy(zeros_hbm, bounce)
    pltpu.sync_copy(bounce, agg_sh.at[pl.ds(r0, ROWS_PER_TILE)])
    pltpu.sync_copy(src2_hbm.at[pl.ds(w * NCHUNK, NCHUNK)], sidx)
    pltpu.sync_copy(dst2_hbm.at[pl.ds(w * NCHUNK, NCHUNK)], didx)
    plsc.subcore_barrier()

    pltpu.async_copy(y_hbm.at[sidx.at[0]], rows0, sem0)

    def body(i, carry):
        j0 = 2 * i
        j1 = 2 * i + 1
        j2 = 2 * i + 2
        pltpu.make_async_copy(y_hbm.at[sidx.at[j0]], rows0, sem0).wait()
        pltpu.async_copy(y_hbm.at[sidx.at[j1]], rows1, sem1)
        pltpu.sync_copy(rows0, agg_sh.at[didx.at[j0]], add=True)
        pltpu.make_async_copy(y_hbm.at[sidx.at[j1]], rows1, sem1).wait()

        @pl.when(j2 < NCHUNK)
        def _():
            pltpu.async_copy(y_hbm.at[sidx.at[j2]], rows0, sem0)

        pltpu.sync_copy(rows1, agg_sh.at[didx.at[j1]], add=True)
        return carry

    lax.fori_loop(0, NCHUNK // 2, body, 0)
    plsc.subcore_barrier()
    pltpu.sync_copy(agg_sh.at[pl.ds(r0, ROWS_PER_TILE)], bounce)
    pltpu.sync_copy(bounce, agg_out.at[c, pl.ds(r0, ROWS_PER_TILE)])


@functools.cache
def _sc_kernels():
    mesh = plsc.VectorSubcoreMesh(
        core_axis_name="c", subcore_axis_name="s", num_cores=NC, num_subcores=NS
    )
    params = pltpu.CompilerParams(use_tc_tiling_on_sc=False)
    deg_kernel = pl.kernel(
        _deg_body,
        out_type=jax.ShapeDtypeStruct((NC * NPAD,), jnp.float32),
        mesh=mesh,
        scratch_types=[
            pltpu.VMEM_SHARED((NPAD,), jnp.float32),
            pltpu.VMEM((NCHUNK, CH), jnp.int32),
            pltpu.VMEM((CH,), jnp.float32),
            pltpu.VMEM((ROWS_PER_TILE,), jnp.float32),
            pltpu.SemaphoreType.DMA,
        ],
        compiler_params=params,
    )
    agg_kernel = pl.kernel(
        _agg_body,
        out_type=jax.ShapeDtypeStruct((NC, NPAD, H), jnp.float32),
        mesh=mesh,
        scratch_types=[
            pltpu.VMEM_SHARED((NPAD, H), jnp.float32),
            pltpu.VMEM((NCHUNK, CH), jnp.int32),
            pltpu.VMEM((NCHUNK, CH), jnp.int32),
            pltpu.VMEM((CH, H), jnp.float32),
            pltpu.VMEM((CH, H), jnp.float32),
            pltpu.VMEM((ROWS_PER_TILE, H), jnp.float32),
            pltpu.SemaphoreType.DMA,
            pltpu.SemaphoreType.DMA,
        ],
        compiler_params=params,
    )
    return deg_kernel, agg_kernel


_PREC = lax.Precision.HIGHEST


def _tc1_body(x_ref, w1_ref, d0_ref, d1_ref, y1_ref, dinv_ref):
    deg = d0_ref[...] + d1_ref[...] + 1.0
    dinv = lax.rsqrt(deg)
    xw = jnp.dot(x_ref[...], w1_ref[...], precision=_PREC,
                 preferred_element_type=jnp.float32)
    dn = dinv[:N]
    y1_ref[...] = xw * dn
    dinv_ref[...] = dn


def _tc2_body(aggp_ref, y1_ref, dinv_ref, b1_ref, w2_ref, y2_ref):
    aggp = aggp_ref[...]
    agg = aggp[0] + aggp[1]
    t = dinv_ref[...] * (agg[:N] + y1_ref[...]) + b1_ref[...]
    h = jnp.maximum(t, 0.0)
    y2_ref[...] = dinv_ref[...] * jnp.dot(
        h, w2_ref[...], precision=_PREC, preferred_element_type=jnp.float32
    )


def _tc3_body(aggp_ref, y2_ref, dinv_ref, b2_ref, batch_ref, out_ref):
    aggp = aggp_ref[...]
    agg = aggp[0] + aggp[1]
    t = dinv_ref[...] * (agg[:N] + y2_ref[...])
    h2 = t[:, :C] + b2_ref[...]
    b = batch_ref[...]
    gids = lax.broadcasted_iota(jnp.int32, (G, 1), 0)
    onehot = (b == gids).astype(jnp.float32)
    sums = jnp.dot(onehot, h2, precision=_PREC,
                   preferred_element_type=jnp.float32)
    counts = jnp.sum(onehot, axis=1, keepdims=True)
    pooled = sums / jnp.maximum(counts, 1.0)
    m = jnp.max(pooled, axis=1, keepdims=True)
    lse = m + jnp.log(jnp.sum(jnp.exp(pooled - m), axis=1, keepdims=True))
    out_ref[...] = pooled - lse


_tc1 = pl.pallas_call(
    _tc1_body,
    out_shape=[
        jax.ShapeDtypeStruct((N, H), jnp.float32),
        jax.ShapeDtypeStruct((N, 1), jnp.float32),
    ],
)

_tc2 = pl.pallas_call(
    _tc2_body,
    out_shape=jax.ShapeDtypeStruct((N, H), jnp.float32),
)

_tc3 = pl.pallas_call(
    _tc3_body,
    out_shape=jax.ShapeDtypeStruct((G, C), jnp.float32),
)


def kernel(x, edge_index, batch, W1, b1, W2, b2):
    src = edge_index[0]
    dst = edge_index[1]
    pad_e = EPAD - E
    pad_src = (jnp.arange(pad_e, dtype=jnp.int32) * 37) % N
    pad_dst = N + (jnp.arange(pad_e, dtype=jnp.int32) % (NPAD - N))
    src2 = jnp.concatenate([src, pad_src]).reshape(NW * NCHUNK, CH)
    dst2 = jnp.concatenate([dst, pad_dst]).reshape(NW * NCHUNK, CH)
    zeros1 = jnp.zeros((ROWS_PER_TILE,), jnp.float32)
    zeros2 = jnp.zeros((ROWS_PER_TILE, H), jnp.float32)
    w2pad = jnp.concatenate([W2, jnp.zeros((H, H - C), jnp.float32)], axis=1)

    _deg_kernel, _agg_kernel = _sc_kernels()
    degp = _deg_kernel(dst2, zeros1)
    d0 = degp[:NPAD].reshape(NPAD, 1)
    d1 = degp[NPAD:].reshape(NPAD, 1)
    y1, dinv = _tc1(x, W1, d0, d1)
    agg1 = _agg_kernel(y1, src2, dst2, zeros2)
    y2 = _tc2(agg1, y1, dinv, b1.reshape(1, H), w2pad)
    agg2 = _agg_kernel(y2, src2, dst2, zeros2)
    out = _tc3(agg2, y2, dinv, b2.reshape(1, C), batch.reshape(1, N))
    return out


# speedup vs baseline: 55.4694x; 1.4559x over previous
import functools

import jax
import jax.numpy as jnp
from jax import lax
from jax.experimental import pallas as pl
from jax.experimental.pallas import tpu as pltpu
from jax.experimental.pallas import tpu_sc as plsc

N, E, D, H, C, G = 10000, 320000, 128, 16, 8, 64
NC, NS = 2, 16
NW = NC * NS
CH = 128
NPAD = 10112
NCHUNK = 80
EPW = NCHUNK * CH
ROWS_PER_TILE = NPAD // NS
EPAD = EPW * NW

def _deg_body(dst2_hbm, zeros_hbm, deg_out, deg_sh, didx, ones_v, bounce, dsem):
    c = lax.axis_index("c")
    s = lax.axis_index("s")
    w = c * NS + s
    r0 = s * ROWS_PER_TILE
    pltpu.sync_copy(zeros_hbm, bounce)
    pltpu.sync_copy(bounce, deg_sh.at[pl.ds(r0, ROWS_PER_TILE)])
    pltpu.sync_copy(dst2_hbm.at[pl.ds(w * NCHUNK, NCHUNK)], didx)
    for i in range(CH // 16):
        ones_v[pl.ds(i * 16, 16)] = jnp.full((16,), 1.0, jnp.float32)
    plsc.subcore_barrier()

    def body(g, carry):
        j0 = g * 4
        for k in range(4):
            pltpu.async_copy(ones_v, deg_sh.at[didx.at[j0 + k]], dsem, add=True)
        for k in range(4):
            pltpu.make_async_copy(ones_v, deg_sh.at[didx.at[j0 + k]], dsem).wait()
        return carry

    lax.fori_loop(0, NCHUNK // 4, body, 0)
    plsc.subcore_barrier()
    pltpu.sync_copy(deg_sh.at[pl.ds(r0, ROWS_PER_TILE)], bounce)
    pltpu.sync_copy(bounce, deg_out.at[pl.ds(c * NPAD + r0, ROWS_PER_TILE)])


NBUF = 4


def _agg_body(y_hbm, src2_hbm, dst2_hbm, zeros_hbm, agg_out, agg_sh, sidx, didx,
              rows, bounce, gsems, ssems):
    c = lax.axis_index("c")
    s = lax.axis_index("s")
    w = c * NS + s
    r0 = s * ROWS_PER_TILE
    pltpu.sync_copy(zeros_hbm, bounce)
    pltpu.sync_copy(bounce, agg_sh.at[pl.ds(r0, ROWS_PER_TILE)])
    pltpu.sync_copy(src2_hbm.at[pl.ds(w * NCHUNK, NCHUNK)], sidx)
    pltpu.sync_copy(dst2_hbm.at[pl.ds(w * NCHUNK, NCHUNK)], didx)
    plsc.subcore_barrier()

    for k in range(NBUF):
        pltpu.async_copy(y_hbm.at[sidx.at[k]], rows[k], gsems[k])

    def body(i, carry):
        for k in range(NBUF):
            j = i * NBUF + k
            jn = j + NBUF
            pltpu.make_async_copy(y_hbm.at[sidx.at[j]], rows[k], gsems[k]).wait()
            pltpu.async_copy(rows[k], agg_sh.at[didx.at[j]], ssems[k], add=True)
            pltpu.make_async_copy(rows[k], agg_sh.at[didx.at[j]], ssems[k]).wait()

            @pl.when(jn < NCHUNK)
            def _():
                pltpu.async_copy(y_hbm.at[sidx.at[jn]], rows[k], gsems[k])

        return carry

    lax.fori_loop(0, NCHUNK // NBUF, body, 0)
    plsc.subcore_barrier()
    pltpu.sync_copy(agg_sh.at[pl.ds(r0, ROWS_PER_TILE)], bounce)
    pltpu.sync_copy(bounce, agg_out.at[c, pl.ds(r0, ROWS_PER_TILE)])


@functools.cache
def _sc_kernels():
    mesh = plsc.VectorSubcoreMesh(
        core_axis_name="c", subcore_axis_name="s", num_cores=NC, num_subcores=NS
    )
    params = pltpu.CompilerParams(use_tc_tiling_on_sc=False)
    deg_kernel = pl.kernel(
        _deg_body,
        out_type=jax.ShapeDtypeStruct((NC * NPAD,), jnp.float32),
        mesh=mesh,
        scratch_types=[
            pltpu.VMEM_SHARED((NPAD,), jnp.float32),
            pltpu.VMEM((NCHUNK, CH), jnp.int32),
            pltpu.VMEM((CH,), jnp.float32),
            pltpu.VMEM((ROWS_PER_TILE,), jnp.float32),
            pltpu.SemaphoreType.DMA,
        ],
        compiler_params=params,
    )
    agg_kernel = pl.kernel(
        _agg_body,
        out_type=jax.ShapeDtypeStruct((NC, NPAD, H), jnp.float32),
        mesh=mesh,
        scratch_types=[
            pltpu.VMEM_SHARED((NPAD, H), jnp.float32),
            pltpu.VMEM((NCHUNK, CH), jnp.int32),
            pltpu.VMEM((NCHUNK, CH), jnp.int32),
            [pltpu.VMEM((CH, H), jnp.float32) for _ in range(NBUF)],
            pltpu.VMEM((ROWS_PER_TILE, H), jnp.float32),
            [pltpu.SemaphoreType.DMA for _ in range(NBUF)],
            [pltpu.SemaphoreType.DMA for _ in range(NBUF)],
        ],
        compiler_params=params,
    )
    return deg_kernel, agg_kernel


_PREC = lax.Precision.HIGHEST


def _tc1_body(x_ref, w1_ref, d0_ref, d1_ref, y1_ref, dinv_ref):
    deg = d0_ref[...] + d1_ref[...] + 1.0
    dinv = lax.rsqrt(deg)
    xw = jnp.dot(x_ref[...], w1_ref[...], precision=_PREC,
                 preferred_element_type=jnp.float32)
    dn = dinv[:N]
    y1_ref[...] = xw * dn
    dinv_ref[...] = dn


def _tc2_body(aggp_ref, y1_ref, dinv_ref, b1_ref, w2_ref, y2_ref):
    aggp = aggp_ref[...]
    agg = aggp[0] + aggp[1]
    t = dinv_ref[...] * (agg[:N] + y1_ref[...]) + b1_ref[...]
    h = jnp.maximum(t, 0.0)
    y2_ref[...] = dinv_ref[...] * jnp.dot(
        h, w2_ref[...], precision=_PREC, preferred_element_type=jnp.float32
    )


def _tc3_body(aggp_ref, y2_ref, dinv_ref, b2_ref, batch_ref, out_ref):
    aggp = aggp_ref[...]
    agg = aggp[0] + aggp[1]
    t = dinv_ref[...] * (agg[:N] + y2_ref[...])
    h2 = t[:, :C] + b2_ref[...]
    b = batch_ref[...]
    gids = lax.broadcasted_iota(jnp.int32, (G, 1), 0)
    onehot = (b == gids).astype(jnp.float32)
    sums = jnp.dot(onehot, h2, precision=_PREC,
                   preferred_element_type=jnp.float32)
    counts = jnp.sum(onehot, axis=1, keepdims=True)
    pooled = sums / jnp.maximum(counts, 1.0)
    m = jnp.max(pooled, axis=1, keepdims=True)
    lse = m + jnp.log(jnp.sum(jnp.exp(pooled - m), axis=1, keepdims=True))
    out_ref[...] = pooled - lse


_tc1 = pl.pallas_call(
    _tc1_body,
    out_shape=[
        jax.ShapeDtypeStruct((N, H), jnp.float32),
        jax.ShapeDtypeStruct((N, 1), jnp.float32),
    ],
)

_tc2 = pl.pallas_call(
    _tc2_body,
    out_shape=jax.ShapeDtypeStruct((N, H), jnp.float32),
)

_tc3 = pl.pallas_call(
    _tc3_body,
    out_shape=jax.ShapeDtypeStruct((G, C), jnp.float32),
)


def kernel(x, edge_index, batch, W1, b1, W2, b2):
    src = edge_index[0]
    dst = edge_index[1]
    pad_e = EPAD - E
    pad_src = (jnp.arange(pad_e, dtype=jnp.int32) * 37) % N
    pad_dst = N + (jnp.arange(pad_e, dtype=jnp.int32) % (NPAD - N))
    src2 = jnp.concatenate([src, pad_src]).reshape(NW * NCHUNK, CH)
    dst2 = jnp.concatenate([dst, pad_dst]).reshape(NW * NCHUNK, CH)
    zeros1 = jnp.zeros((ROWS_PER_TILE,), jnp.float32)
    zeros2 = jnp.zeros((ROWS_PER_TILE, H), jnp.float32)
    w2pad = jnp.concatenate([W2, jnp.zeros((H, H - C), jnp.float32)], axis=1)

    _deg_kernel, _agg_kernel = _sc_kernels()
    degp = _deg_kernel(dst2, zeros1)
    d0 = degp[:NPAD].reshape(NPAD, 1)
    d1 = degp[NPAD:].reshape(NPAD, 1)
    y1, dinv = _tc1(x, W1, d0, d1)
    agg1 = _agg_kernel(y1, src2, dst2, zeros2)
    y2 = _tc2(agg1, y1, dinv, b1.reshape(1, H), w2pad)
    agg2 = _agg_kernel(y2, src2, dst2, zeros2)
    out = _tc3(agg2, y2, dinv, b2.reshape(1, C), batch.reshape(1, N))
    return out


# speedup vs baseline: 60.5080x; 1.0908x over previous
import functools

import jax
import jax.numpy as jnp
from jax import lax
from jax.experimental import pallas as pl
from jax.experimental.pallas import tpu as pltpu
from jax.experimental.pallas import tpu_sc as plsc

N, E, D, H, C, G = 10000, 320000, 128, 16, 8, 64
NC, NS = 2, 16
NW = NC * NS
CH = 128
NPAD = 10112
NCHUNK = 80
EPW = NCHUNK * CH
ROWS_PER_TILE = NPAD // NS
EPAD = EPW * NW

def _deg_body(dst2_hbm, zeros_hbm, deg_out, deg_sh, didx, ones_v, bounce, dsem):
    c = lax.axis_index("c")
    s = lax.axis_index("s")
    w = c * NS + s
    r0 = s * ROWS_PER_TILE
    pltpu.sync_copy(zeros_hbm, bounce)
    pltpu.sync_copy(bounce, deg_sh.at[pl.ds(r0, ROWS_PER_TILE)])
    pltpu.sync_copy(dst2_hbm.at[pl.ds(w * NCHUNK, NCHUNK)], didx)
    for i in range(CH // 16):
        ones_v[pl.ds(i * 16, 16)] = jnp.full((16,), 1.0, jnp.float32)
    plsc.subcore_barrier()

    def body(g, carry):
        j0 = g * 4
        for k in range(4):
            pltpu.async_copy(ones_v, deg_sh.at[didx.at[j0 + k]], dsem, add=True)
        for k in range(4):
            pltpu.make_async_copy(ones_v, deg_sh.at[didx.at[j0 + k]], dsem).wait()
        return carry

    lax.fori_loop(0, NCHUNK // 4, body, 0)
    plsc.subcore_barrier()
    pltpu.sync_copy(deg_sh.at[pl.ds(r0, ROWS_PER_TILE)], bounce)
    pltpu.sync_copy(bounce, deg_out.at[pl.ds(c * NPAD + r0, ROWS_PER_TILE)])


NBUF = 8


def _agg_body(y_hbm, src2_hbm, dst2_hbm, zeros_hbm, agg_out, agg_sh, sidx, didx,
              rows, bounce, gsems, ssems):
    c = lax.axis_index("c")
    s = lax.axis_index("s")
    w = c * NS + s
    r0 = s * ROWS_PER_TILE
    pltpu.sync_copy(zeros_hbm, bounce)
    pltpu.sync_copy(bounce, agg_sh.at[pl.ds(r0, ROWS_PER_TILE)])
    pltpu.sync_copy(src2_hbm.at[pl.ds(w * NCHUNK, NCHUNK)], sidx)
    pltpu.sync_copy(dst2_hbm.at[pl.ds(w * NCHUNK, NCHUNK)], didx)
    plsc.subcore_barrier()

    for k in range(NBUF):
        pltpu.async_copy(y_hbm.at[sidx.at[k]], rows[k], gsems[k])

    def body(i, carry):
        for k in range(NBUF):
            j = i * NBUF + k
            jn = j + NBUF
            pltpu.make_async_copy(y_hbm.at[sidx.at[j]], rows[k], gsems[k]).wait()
            pltpu.async_copy(rows[k], agg_sh.at[didx.at[j]], ssems[k], add=True)
            pltpu.make_async_copy(rows[k], agg_sh.at[didx.at[j]], ssems[k]).wait()

            @pl.when(jn < NCHUNK)
            def _():
                pltpu.async_copy(y_hbm.at[sidx.at[jn]], rows[k], gsems[k])

        return carry

    lax.fori_loop(0, NCHUNK // NBUF, body, 0)
    plsc.subcore_barrier()
    pltpu.sync_copy(agg_sh.at[pl.ds(r0, ROWS_PER_TILE)], bounce)
    pltpu.sync_copy(bounce, agg_out.at[c, pl.ds(r0, ROWS_PER_TILE)])


@functools.cache
def _sc_kernels():
    mesh = plsc.VectorSubcoreMesh(
        core_axis_name="c", subcore_axis_name="s", num_cores=NC, num_subcores=NS
    )
    params = pltpu.CompilerParams(use_tc_tiling_on_sc=False)
    deg_kernel = pl.kernel(
        _deg_body,
        out_type=jax.ShapeDtypeStruct((NC * NPAD,), jnp.float32),
        mesh=mesh,
        scratch_types=[
            pltpu.VMEM_SHARED((NPAD,), jnp.float32),
            pltpu.VMEM((NCHUNK, CH), jnp.int32),
            pltpu.VMEM((CH,), jnp.float32),
            pltpu.VMEM((ROWS_PER_TILE,), jnp.float32),
            pltpu.SemaphoreType.DMA,
        ],
        compiler_params=params,
    )
    agg_kernel = pl.kernel(
        _agg_body,
        out_type=jax.ShapeDtypeStruct((NC, NPAD, H), jnp.float32),
        mesh=mesh,
        scratch_types=[
            pltpu.VMEM_SHARED((NPAD, H), jnp.float32),
            pltpu.VMEM((NCHUNK, CH), jnp.int32),
            pltpu.VMEM((NCHUNK, CH), jnp.int32),
            [pltpu.VMEM((CH, H), jnp.float32) for _ in range(NBUF)],
            pltpu.VMEM((ROWS_PER_TILE, H), jnp.float32),
            [pltpu.SemaphoreType.DMA for _ in range(NBUF)],
            [pltpu.SemaphoreType.DMA for _ in range(NBUF)],
        ],
        compiler_params=params,
    )
    return deg_kernel, agg_kernel


_PREC = lax.Precision.HIGHEST


def _tc1_body(x_ref, w1_ref, d0_ref, d1_ref, y1_ref, dinv_ref):
    deg = d0_ref[...] + d1_ref[...] + 1.0
    dinv = lax.rsqrt(deg)
    xw = jnp.dot(x_ref[...], w1_ref[...], precision=_PREC,
                 preferred_element_type=jnp.float32)
    dn = dinv[:N]
    y1_ref[...] = xw * dn
    dinv_ref[...] = dn


def _tc2_body(aggp_ref, y1_ref, dinv_ref, b1_ref, w2_ref, y2_ref):
    aggp = aggp_ref[...]
    agg = aggp[0] + aggp[1]
    t = dinv_ref[...] * (agg[:N] + y1_ref[...]) + b1_ref[...]
    h = jnp.maximum(t, 0.0)
    y2_ref[...] = dinv_ref[...] * jnp.dot(
        h, w2_ref[...], precision=_PREC, preferred_element_type=jnp.float32
    )


def _tc3_body(aggp_ref, y2_ref, dinv_ref, b2_ref, batch_ref, out_ref):
    aggp = aggp_ref[...]
    agg = aggp[0] + aggp[1]
    t = dinv_ref[...] * (agg[:N] + y2_ref[...])
    h2 = t[:, :C] + b2_ref[...]
    b = batch_ref[...]
    gids = lax.broadcasted_iota(jnp.int32, (G, 1), 0)
    onehot = (b == gids).astype(jnp.float32)
    sums = jnp.dot(onehot, h2, precision=_PREC,
                   preferred_element_type=jnp.float32)
    counts = jnp.sum(onehot, axis=1, keepdims=True)
    pooled = sums / jnp.maximum(counts, 1.0)
    m = jnp.max(pooled, axis=1, keepdims=True)
    lse = m + jnp.log(jnp.sum(jnp.exp(pooled - m), axis=1, keepdims=True))
    out_ref[...] = pooled - lse


_tc1 = pl.pallas_call(
    _tc1_body,
    out_shape=[
        jax.ShapeDtypeStruct((N, H), jnp.float32),
        jax.ShapeDtypeStruct((N, 1), jnp.float32),
    ],
)

_tc2 = pl.pallas_call(
    _tc2_body,
    out_shape=jax.ShapeDtypeStruct((N, H), jnp.float32),
)

_tc3 = pl.pallas_call(
    _tc3_body,
    out_shape=jax.ShapeDtypeStruct((G, C), jnp.float32),
)


def kernel(x, edge_index, batch, W1, b1, W2, b2):
    src = edge_index[0]
    dst = edge_index[1]
    pad_e = EPAD - E
    pad_src = (jnp.arange(pad_e, dtype=jnp.int32) * 37) % N
    pad_dst = N + (jnp.arange(pad_e, dtype=jnp.int32) % (NPAD - N))
    src2 = jnp.concatenate([src, pad_src]).reshape(NW * NCHUNK, CH)
    dst2 = jnp.concatenate([dst, pad_dst]).reshape(NW * NCHUNK, CH)
    zeros1 = jnp.zeros((ROWS_PER_TILE,), jnp.float32)
    zeros2 = jnp.zeros((ROWS_PER_TILE, H), jnp.float32)
    w2pad = jnp.concatenate([W2, jnp.zeros((H, H - C), jnp.float32)], axis=1)

    _deg_kernel, _agg_kernel = _sc_kernels()
    degp = _deg_kernel(dst2, zeros1)
    d0 = degp[:NPAD].reshape(NPAD, 1)
    d1 = degp[NPAD:].reshape(NPAD, 1)
    y1, dinv = _tc1(x, W1, d0, d1)
    agg1 = _agg_kernel(y1, src2, dst2, zeros2)
    y2 = _tc2(agg1, y1, dinv, b1.reshape(1, H), w2pad)
    agg2 = _agg_kernel(y2, src2, dst2, zeros2)
    out = _tc3(agg2, y2, dinv, b2.reshape(1, C), batch.reshape(1, N))
    return out
